# baseline (speedup 1.0000x reference)
import functools

import jax
import jax.numpy as jnp
from jax import lax
from jax.experimental import pallas as pl
from jax.experimental.pallas import tpu as pltpu
from jax.experimental.pallas import tpu_sc as plsc

N = 10000
E = 320000
F_IN = 128
HID = 64
CLS = 47
K = 10
ALPHA = 0.1

NPAD = 10240
CPAD = 48
NTILES = 32
BATCH = 128
EPT = 10240
NB = EPT // BATCH
ROWS_PER_SUB = NPAD // 16

_MESH = plsc.VectorSubcoreMesh(core_axis_name="c", subcore_axis_name="s")
_SC_PARAMS = pltpu.CompilerParams(use_tc_tiling_on_sc=False)


def _fill_f32(ref, value, total):
    vec = jnp.full((16,), value, dtype=jnp.float32)

    def body(i, _):
        ref[pl.ds(i * 16, 16)] = vec
        return 0

    lax.fori_loop(0, total // 16, body, 0)


@functools.partial(
    pl.kernel,
    out_type=jax.ShapeDtypeStruct((2, NPAD), jnp.float32),
    mesh=_MESH,
    scratch_types=[
        pltpu.VMEM((NB, BATCH), jnp.int32),
        pltpu.VMEM((BATCH,), jnp.float32),
        pltpu.VMEM((ROWS_PER_SUB,), jnp.float32),
        pltpu.VMEM_SHARED((NPAD,), jnp.float32),
        pltpu.SemaphoreType.DMA,
    ],
    compiler_params=_SC_PARAMS,
)
def _deg_kernel(dst_hbm, out_hbm, idx_v, ones_v, zeros_v, acc_sh, sem):
    cid = lax.axis_index("c")
    sid = lax.axis_index("s")
    wid = cid * 16 + sid
    _fill_f32(zeros_v, 0.0, ROWS_PER_SUB)
    _fill_f32(ones_v, 1.0, BATCH)
    pltpu.sync_copy(zeros_v, acc_sh.at[pl.ds(sid * ROWS_PER_SUB, ROWS_PER_SUB)])
    plsc.subcore_barrier()
    pltpu.async_copy(dst_hbm.at[wid], idx_v, sem).wait()

    def body(j, _):
        pltpu.sync_copy(ones_v, acc_sh.at[idx_v.at[j]], add=True)
        return 0

    lax.fori_loop(0, NB, body, 0)
    plsc.subcore_barrier()
    sl = pl.ds(sid * ROWS_PER_SUB, ROWS_PER_SUB)
    pltpu.sync_copy(acc_sh.at[sl], out_hbm.at[cid, sl])


@functools.partial(
    pl.kernel,
    out_type=jax.ShapeDtypeStruct((2, NPAD, CPAD), jnp.float32),
    mesh=_MESH,
    scratch_types=[
        pltpu.VMEM((NB, BATCH), jnp.int32),
        pltpu.VMEM((NB, BATCH), jnp.int32),
        [pltpu.VMEM((BATCH, CPAD), jnp.float32)] * 8,
        pltpu.VMEM((BATCH, CPAD), jnp.float32),
        pltpu.VMEM_SHARED((NPAD, CPAD), jnp.float32),
        [pltpu.SemaphoreType.DMA] * 8,
        [pltpu.SemaphoreType.DMA] * 8,
        pltpu.SemaphoreType.DMA,
    ],
    compiler_params=_SC_PARAMS,
)
def _prop_kernel(cur_hbm, src_hbm, dst_hbm, out_hbm,
                 src_v, dst_v, rows, zeros_v, acc_sh,
                 gsems, ssems, semi):
    cid = lax.axis_index("c")
    sid = lax.axis_index("s")
    wid = cid * 16 + sid

    def zfill(r, _):
        zeros_v[r, pl.ds(0, 16)] = jnp.zeros((16,), jnp.float32)
        zeros_v[r, pl.ds(16, 16)] = jnp.zeros((16,), jnp.float32)
        zeros_v[r, pl.ds(32, 16)] = jnp.zeros((16,), jnp.float32)
        return 0

    lax.fori_loop(0, BATCH, zfill, 0)
    base = sid * ROWS_PER_SUB
    for t in range(ROWS_PER_SUB // BATCH):
        pltpu.sync_copy(zeros_v, acc_sh.at[pl.ds(base + t * BATCH, BATCH)])

    pltpu.async_copy(src_hbm.at[wid], src_v, semi).wait()
    pltpu.async_copy(dst_hbm.at[wid], dst_v, semi).wait()
    plsc.subcore_barrier()

    for j in range(4):
        pltpu.async_copy(cur_hbm.at[src_v.at[j]], rows[j], gsems[j])

    def body(jj, _):
        for b in range(8):
            j = jj * 8 + b
            pltpu.make_async_copy(cur_hbm.at[src_v.at[j]], rows[b], gsems[b]).wait()
            pltpu.async_copy(rows[b], acc_sh.at[dst_v.at[j]], ssems[b], add=True)
            @pl.when(j + 4 < NB)
            def _():
                bn = (b + 4) % 8
                @pl.when(j >= 4)
                def _():
                    pltpu.make_async_copy(
                        rows[bn], acc_sh.at[dst_v.at[j - 4]], ssems[bn]).wait()
                pltpu.async_copy(cur_hbm.at[src_v.at[j + 4]], rows[bn], gsems[bn])
        return 0

    lax.fori_loop(0, NB // 8, body, 0)
    for j in range(NB - 8, NB):
        b = j % 8
        pltpu.make_async_copy(rows[b], acc_sh.at[dst_v.at[j]], ssems[b]).wait()
    plsc.subcore_barrier()
    sl = pl.ds(sid * ROWS_PER_SUB, ROWS_PER_SUB)
    pltpu.sync_copy(acc_sh.at[sl], out_hbm.at[cid, sl])


def _mlp_body(x_ref, w1_ref, b1_ref, w2_ref, b2_ref, deg_ref,
              u0_ref, dinvsq_ref):
    h = jnp.maximum(
        jnp.dot(x_ref[...], w1_ref[...], preferred_element_type=jnp.float32)
        + b1_ref[...][None, :], 0.0)
    h = jnp.dot(h, w2_ref[...], preferred_element_type=jnp.float32) \
        + b2_ref[...][None, :]
    deg = deg_ref[0, :] + deg_ref[1, :] + 1.0
    dinv = lax.rsqrt(deg)
    u0_ref[...] = h * dinv[:, None]
    dinvsq_ref[...] = 1.0 / deg


def _mlp(x_pad, W1, b1, W2p, b2p, deg_part):
    blk = 512
    grid = NPAD // blk
    return pl.pallas_call(
        _mlp_body,
        grid=(grid,),
        in_specs=[
            pl.BlockSpec((blk, F_IN), lambda i: (i, 0)),
            pl.BlockSpec((F_IN, HID), lambda i: (0, 0)),
            pl.BlockSpec((HID,), lambda i: (0,)),
            pl.BlockSpec((HID, CPAD), lambda i: (0, 0)),
            pl.BlockSpec((CPAD,), lambda i: (0,)),
            pl.BlockSpec((2, blk), lambda i: (0, i)),
        ],
        out_specs=[
            pl.BlockSpec((blk, CPAD), lambda i: (i, 0)),
            pl.BlockSpec((blk,), lambda i: (i,)),
        ],
        out_shape=[
            jax.ShapeDtypeStruct((NPAD, CPAD), jnp.float32),
            jax.ShapeDtypeStruct((NPAD,), jnp.float32),
        ],
    )(x_pad, W1, b1, W2p, b2p, deg_part)


def _combine_body(part_ref, u_ref, dinvsq_ref, out_ref):
    s = part_ref[0] + part_ref[1] + u_ref[...]
    out_ref[...] = s * dinvsq_ref[...][:, None]


def _combine(part, u, dinvsq):
    blk = 512
    grid = NPAD // blk
    return pl.pallas_call(
        _combine_body,
        grid=(grid,),
        in_specs=[
            pl.BlockSpec((2, blk, CPAD), lambda i: (0, i, 0)),
            pl.BlockSpec((blk, CPAD), lambda i: (i, 0)),
            pl.BlockSpec((blk,), lambda i: (i,)),
        ],
        out_specs=pl.BlockSpec((blk, CPAD), lambda i: (i, 0)),
        out_shape=jax.ShapeDtypeStruct((NPAD, CPAD), jnp.float32),
    )(part, u, dinvsq)


def _final_body(coefs, *refs):
    us = refs[:-2]
    dinvsq_ref = refs[-2]
    out_ref = refs[-1]
    acc = coefs[0] * us[0][...]
    for c, u in zip(coefs[1:], us[1:]):
        acc = acc + c * u[...]
    v = acc * lax.rsqrt(dinvsq_ref[...])
    col = lax.broadcasted_iota(jnp.int32, v.shape, 1)
    valid = col < CLS
    neg = jnp.full_like(v, -jnp.inf)
    m = jnp.max(jnp.where(valid, v, neg), axis=1, keepdims=True)
    ex = jnp.where(valid, jnp.exp(v - m), 0.0)
    s = jnp.sum(ex, axis=1, keepdims=True)
    res = v - m - jnp.log(s)
    out_ref[...] = res[:, :CLS]


def _final(us, dinvsq, coefs):
    blk = 400
    grid = N // blk
    return pl.pallas_call(
        functools.partial(_final_body, coefs),
        grid=(grid,),
        in_specs=[pl.BlockSpec((blk, CPAD), lambda i: (i, 0)) for _ in us]
        + [pl.BlockSpec((blk, 1), lambda i: (i, 0))],
        out_specs=pl.BlockSpec((blk, CLS), lambda i: (i, 0)),
        out_shape=jax.ShapeDtypeStruct((N, CLS), jnp.float32),
    )(*us, dinvsq[:, None])


def kernel(x, edge_index, W1, b1, W2, b2):
    src = edge_index[0].astype(jnp.int32)
    dst = edge_index[1].astype(jnp.int32)
    epad = NTILES * EPT - E
    src = jnp.concatenate([src, jnp.zeros((epad,), jnp.int32)])
    dst = jnp.concatenate([dst, jnp.full((epad,), NPAD - 1, jnp.int32)])
    src_t = src.reshape(NTILES, NB, BATCH)
    dst_t = dst.reshape(NTILES, NB, BATCH)

    x_pad = jnp.pad(x, ((0, NPAD - N), (0, 0)))
    W2p = jnp.pad(W2, ((0, 0), (0, CPAD - CLS)))
    b2p = jnp.pad(b2, ((0, CPAD - CLS),))

    deg_part = _deg_kernel(dst_t)
    u, dinvsq = _mlp(x_pad, W1, b1, W2p, b2p, deg_part)

    khalf = K // 2
    coef = [ALPHA * (1.0 - ALPHA) ** i for i in range(khalf + 1)]
    coef[khalf] = (1.0 - ALPHA) ** khalf

    evens = [u]
    for _ in range(khalf):
        for _ in range(2):
            part = _prop_kernel(u, src_t, dst_t)
            u = _combine(part, u, dinvsq)
        evens.append(u)

    return _final(evens, dinvsq, coef)


# speedup vs baseline: 2.1686x; 2.1686x over previous
import functools

import jax
import jax.numpy as jnp
from jax import lax
from jax.experimental import pallas as pl
from jax.experimental.pallas import tpu as pltpu
from jax.experimental.pallas import tpu_sc as plsc

N = 10000
E = 320000
F_IN = 128
HID = 64
CLS = 47
K = 10
ALPHA = 0.1

NPAD = 10240
CPAD = 48
NTILES = 32
BATCH = 128
EPT = 10240
NB = EPT // BATCH
ROWS_PER_SUB = NPAD // 16

_MESH = plsc.VectorSubcoreMesh(core_axis_name="c", subcore_axis_name="s")
_SC_PARAMS = pltpu.CompilerParams(use_tc_tiling_on_sc=False)


def _fill_f32(ref, value, total):
    vec = jnp.full((16,), value, dtype=jnp.float32)

    def body(i, _):
        ref[pl.ds(i * 16, 16)] = vec
        return 0

    lax.fori_loop(0, total // 16, body, 0)


@functools.partial(
    pl.kernel,
    out_type=jax.ShapeDtypeStruct((2, NPAD), jnp.float32),
    mesh=_MESH,
    scratch_types=[
        pltpu.VMEM((NB, BATCH), jnp.int32),
        pltpu.VMEM((BATCH,), jnp.float32),
        pltpu.VMEM((ROWS_PER_SUB,), jnp.float32),
        pltpu.VMEM_SHARED((NPAD,), jnp.float32),
        pltpu.SemaphoreType.DMA,
    ],
    compiler_params=_SC_PARAMS,
)
def _deg_kernel(dst_hbm, out_hbm, idx_v, ones_v, zeros_v, acc_sh, sem):
    cid = lax.axis_index("c")
    sid = lax.axis_index("s")
    wid = cid * 16 + sid
    _fill_f32(zeros_v, 0.0, ROWS_PER_SUB)
    _fill_f32(ones_v, 1.0, BATCH)
    pltpu.sync_copy(zeros_v, acc_sh.at[pl.ds(sid * ROWS_PER_SUB, ROWS_PER_SUB)])
    plsc.subcore_barrier()
    pltpu.async_copy(dst_hbm.at[wid], idx_v, sem).wait()

    def body(j, _):
        pltpu.sync_copy(ones_v, acc_sh.at[idx_v.at[j]], add=True)
        return 0

    lax.fori_loop(0, NB, body, 0)
    plsc.subcore_barrier()
    sl = pl.ds(sid * ROWS_PER_SUB, ROWS_PER_SUB)
    pltpu.sync_copy(acc_sh.at[sl], out_hbm.at[cid, sl])


CH = CPAD // 2


@functools.partial(
    pl.kernel,
    out_type=jax.ShapeDtypeStruct((2, NPAD, CPAD), jnp.float32),
    mesh=_MESH,
    scratch_types=[
        pltpu.VMEM((NB, BATCH), jnp.int32),
        pltpu.VMEM((NB, BATCH), jnp.int32),
        [pltpu.VMEM((BATCH, CH), jnp.float32)] * 8,
        pltpu.VMEM_SHARED((NPAD, CH), jnp.float32),
        pltpu.VMEM_SHARED((NPAD, CH), jnp.float32),
        [pltpu.SemaphoreType.DMA] * 8,
        [pltpu.SemaphoreType.DMA] * 8,
        pltpu.SemaphoreType.DMA,
    ],
    compiler_params=_SC_PARAMS,
)
def _prop_kernel(cur_hbm, src_hbm, dst_hbm, zeros_hbm, out_hbm,
                 src_v, dst_v, rows, acc_sh, u_sh,
                 gsems, ssems, semi):
    cid = lax.axis_index("c")
    sid = lax.axis_index("s")
    wid = cid * 16 + sid
    base = sid * ROWS_PER_SUB
    sl = pl.ds(base, ROWS_PER_SUB)

    pltpu.async_copy(src_hbm.at[wid], src_v, semi).wait()
    pltpu.async_copy(dst_hbm.at[wid], dst_v, semi).wait()

    for h in range(2):
        ch = pl.ds(h * CH, CH)
        pltpu.sync_copy(zeros_hbm.at[sl], acc_sh.at[sl])
        pltpu.sync_copy(cur_hbm.at[sl, ch], u_sh.at[sl])
        plsc.subcore_barrier()

        for j in range(4):
            pltpu.async_copy(u_sh.at[src_v.at[j]], rows[j], gsems[j])

        def body(jj, _):
            for b in range(8):
                j = jj * 8 + b
                pltpu.make_async_copy(u_sh.at[src_v.at[j]], rows[b], gsems[b]).wait()
                pltpu.async_copy(rows[b], acc_sh.at[dst_v.at[j]], ssems[b], add=True)
                @pl.when(j + 4 < NB)
                def _():
                    bn = (b + 4) % 8
                    @pl.when(j >= 4)
                    def _():
                        pltpu.make_async_copy(
                            rows[bn], acc_sh.at[dst_v.at[j - 4]], ssems[bn]).wait()
                    pltpu.async_copy(u_sh.at[src_v.at[j + 4]], rows[bn], gsems[bn])
            return 0

        lax.fori_loop(0, NB // 8, body, 0)
        for j in range(NB - 8, NB):
            b = j % 8
            pltpu.make_async_copy(rows[b], acc_sh.at[dst_v.at[j]], ssems[b]).wait()
        plsc.subcore_barrier()
        pltpu.sync_copy(acc_sh.at[sl], out_hbm.at[cid, sl, ch])
        plsc.subcore_barrier()


def _mlp_body(x_ref, w1_ref, b1_ref, w2_ref, b2_ref, deg_ref,
              u0_ref, dinvsq_ref):
    h = jnp.maximum(
        jnp.dot(x_ref[...], w1_ref[...], preferred_element_type=jnp.float32)
        + b1_ref[...][None, :], 0.0)
    h = jnp.dot(h, w2_ref[...], preferred_element_type=jnp.float32) \
        + b2_ref[...][None, :]
    deg = deg_ref[0, :] + deg_ref[1, :] + 1.0
    dinv = lax.rsqrt(deg)
    u0_ref[...] = h * dinv[:, None]
    dinvsq_ref[...] = 1.0 / deg


def _mlp(x_pad, W1, b1, W2p, b2p, deg_part):
    blk = 512
    grid = NPAD // blk
    return pl.pallas_call(
        _mlp_body,
        grid=(grid,),
        in_specs=[
            pl.BlockSpec((blk, F_IN), lambda i: (i, 0)),
            pl.BlockSpec((F_IN, HID), lambda i: (0, 0)),
            pl.BlockSpec((HID,), lambda i: (0,)),
            pl.BlockSpec((HID, CPAD), lambda i: (0, 0)),
            pl.BlockSpec((CPAD,), lambda i: (0,)),
            pl.BlockSpec((2, blk), lambda i: (0, i)),
        ],
        out_specs=[
            pl.BlockSpec((blk, CPAD), lambda i: (i, 0)),
            pl.BlockSpec((blk,), lambda i: (i,)),
        ],
        out_shape=[
            jax.ShapeDtypeStruct((NPAD, CPAD), jnp.float32),
            jax.ShapeDtypeStruct((NPAD,), jnp.float32),
        ],
    )(x_pad, W1, b1, W2p, b2p, deg_part)


def _combine_body(part_ref, u_ref, dinvsq_ref, out_ref):
    s = part_ref[0] + part_ref[1] + u_ref[...]
    out_ref[...] = s * dinvsq_ref[...][:, None]


def _combine(part, u, dinvsq):
    blk = 512
    grid = NPAD // blk
    return pl.pallas_call(
        _combine_body,
        grid=(grid,),
        in_specs=[
            pl.BlockSpec((2, blk, CPAD), lambda i: (0, i, 0)),
            pl.BlockSpec((blk, CPAD), lambda i: (i, 0)),
            pl.BlockSpec((blk,), lambda i: (i,)),
        ],
        out_specs=pl.BlockSpec((blk, CPAD), lambda i: (i, 0)),
        out_shape=jax.ShapeDtypeStruct((NPAD, CPAD), jnp.float32),
    )(part, u, dinvsq)


def _final_body(coefs, *refs):
    us = refs[:-2]
    dinvsq_ref = refs[-2]
    out_ref = refs[-1]
    acc = coefs[0] * us[0][...]
    for c, u in zip(coefs[1:], us[1:]):
        acc = acc + c * u[...]
    v = acc * lax.rsqrt(dinvsq_ref[...])
    col = lax.broadcasted_iota(jnp.int32, v.shape, 1)
    valid = col < CLS
    neg = jnp.full_like(v, -jnp.inf)
    m = jnp.max(jnp.where(valid, v, neg), axis=1, keepdims=True)
    ex = jnp.where(valid, jnp.exp(v - m), 0.0)
    s = jnp.sum(ex, axis=1, keepdims=True)
    res = v - m - jnp.log(s)
    out_ref[...] = res[:, :CLS]


def _final(us, dinvsq, coefs):
    blk = 400
    grid = N // blk
    return pl.pallas_call(
        functools.partial(_final_body, coefs),
        grid=(grid,),
        in_specs=[pl.BlockSpec((blk, CPAD), lambda i: (i, 0)) for _ in us]
        + [pl.BlockSpec((blk, 1), lambda i: (i, 0))],
        out_specs=pl.BlockSpec((blk, CLS), lambda i: (i, 0)),
        out_shape=jax.ShapeDtypeStruct((N, CLS), jnp.float32),
    )(*us, dinvsq[:, None])


def kernel(x, edge_index, W1, b1, W2, b2):
    src = edge_index[0].astype(jnp.int32)
    dst = edge_index[1].astype(jnp.int32)
    epad = NTILES * EPT - E
    src = jnp.concatenate([src, jnp.zeros((epad,), jnp.int32)])
    dst = jnp.concatenate([dst, jnp.full((epad,), NPAD - 1, jnp.int32)])
    src_t = src.reshape(NTILES, NB, BATCH)
    dst_t = dst.reshape(NTILES, NB, BATCH)

    x_pad = jnp.pad(x, ((0, NPAD - N), (0, 0)))
    W2p = jnp.pad(W2, ((0, 0), (0, CPAD - CLS)))
    b2p = jnp.pad(b2, ((0, CPAD - CLS),))

    deg_part = _deg_kernel(dst_t)
    u, dinvsq = _mlp(x_pad, W1, b1, W2p, b2p, deg_part)

    zeros_pad = jnp.zeros((NPAD, CH), jnp.float32)

    khalf = K // 2
    coef = [ALPHA * (1.0 - ALPHA) ** i for i in range(khalf + 1)]
    coef[khalf] = (1.0 - ALPHA) ** khalf

    evens = [u]
    for _ in range(khalf):
        for _ in range(2):
            part = _prop_kernel(u, src_t, dst_t, zeros_pad)
            u = _combine(part, u, dinvsq)
        evens.append(u)

    return _final(evens, dinvsq, coef)


# speedup vs baseline: 2.3307x; 1.0748x over previous
import functools

import jax
import jax.numpy as jnp
from jax import lax
from jax.experimental import pallas as pl
from jax.experimental.pallas import tpu as pltpu
from jax.experimental.pallas import tpu_sc as plsc

N = 10000
E = 320000
F_IN = 128
HID = 64
CLS = 47
K = 10
ALPHA = 0.1

NPAD = 10240
CPAD = 48
NTILES = 32
BATCH = 128
EPT = 10240
NB = EPT // BATCH
ROWS_PER_SUB = NPAD // 16

_MESH = plsc.VectorSubcoreMesh(core_axis_name="c", subcore_axis_name="s")
_SC_PARAMS = pltpu.CompilerParams(use_tc_tiling_on_sc=False)


def _fill_f32(ref, value, total):
    vec = jnp.full((16,), value, dtype=jnp.float32)

    def body(i, _):
        ref[pl.ds(i * 16, 16)] = vec
        return 0

    lax.fori_loop(0, total // 16, body, 0)


@functools.partial(
    pl.kernel,
    out_type=jax.ShapeDtypeStruct((2, NPAD), jnp.float32),
    mesh=_MESH,
    scratch_types=[
        pltpu.VMEM((NB, BATCH), jnp.int32),
        pltpu.VMEM((BATCH,), jnp.float32),
        pltpu.VMEM((ROWS_PER_SUB,), jnp.float32),
        pltpu.VMEM_SHARED((NPAD,), jnp.float32),
        pltpu.SemaphoreType.DMA,
    ],
    compiler_params=_SC_PARAMS,
)
def _deg_kernel(dst_hbm, out_hbm, idx_v, ones_v, zeros_v, acc_sh, sem):
    cid = lax.axis_index("c")
    sid = lax.axis_index("s")
    wid = cid * 16 + sid
    _fill_f32(zeros_v, 0.0, ROWS_PER_SUB)
    _fill_f32(ones_v, 1.0, BATCH)
    pltpu.sync_copy(zeros_v, acc_sh.at[pl.ds(sid * ROWS_PER_SUB, ROWS_PER_SUB)])
    plsc.subcore_barrier()
    pltpu.async_copy(dst_hbm.at[wid], idx_v, sem).wait()

    def body(j, _):
        pltpu.sync_copy(ones_v, acc_sh.at[idx_v.at[j]], add=True)
        return 0

    lax.fori_loop(0, NB, body, 0)
    plsc.subcore_barrier()
    sl = pl.ds(sid * ROWS_PER_SUB, ROWS_PER_SUB)
    pltpu.sync_copy(acc_sh.at[sl], out_hbm.at[cid, sl])


CH = CPAD // 2
SLAB = 128
NSLAB = ROWS_PER_SUB // SLAB


def _edge_pipeline(u_sh, acc_sh, src_v, dst_v, rows, gsems, ssems):
    for j in range(4):
        pltpu.async_copy(u_sh.at[src_v.at[j]], rows[j], gsems[j])

    def body(jj, _):
        for b in range(8):
            j = jj * 8 + b
            pltpu.make_async_copy(u_sh.at[src_v.at[j]], rows[b], gsems[b]).wait()
            pltpu.async_copy(rows[b], acc_sh.at[dst_v.at[j]], ssems[b], add=True)
            @pl.when(j + 4 < NB)
            def _():
                bn = (b + 4) % 8
                @pl.when(j >= 4)
                def _():
                    pltpu.make_async_copy(
                        rows[bn], acc_sh.at[dst_v.at[j - 4]], ssems[bn]).wait()
                pltpu.async_copy(u_sh.at[src_v.at[j + 4]], rows[bn], gsems[bn])
        return 0

    lax.fori_loop(0, NB // 8, body, 0)
    for j in range(NB - 8, NB):
        b = j % 8
        pltpu.make_async_copy(rows[b], acc_sh.at[dst_v.at[j]], ssems[b]).wait()


@functools.partial(
    pl.kernel,
    out_type=jax.ShapeDtypeStruct((2, NPAD, CPAD), jnp.float32),
    mesh=_MESH,
    scratch_types=[
        pltpu.VMEM((NB, BATCH), jnp.int32),
        pltpu.VMEM((NB, BATCH), jnp.int32),
        [pltpu.VMEM((BATCH, CH), jnp.float32)] * 8,
        pltpu.VMEM_SHARED((NPAD, CH), jnp.float32),
        pltpu.VMEM_SHARED((NPAD, CH), jnp.float32),
        [pltpu.SemaphoreType.DMA] * 8,
        [pltpu.SemaphoreType.DMA] * 8,
        pltpu.SemaphoreType.DMA,
    ],
    compiler_params=_SC_PARAMS,
)
def _prop_first(cur_hbm, src_hbm, dst_hbm, zeros_hbm, out_hbm,
                src_v, dst_v, rows, acc_sh, u_sh, gsems, ssems, semi):
    cid = lax.axis_index("c")
    sid = lax.axis_index("s")
    wid = cid * 16 + sid
    base = sid * ROWS_PER_SUB
    sl = pl.ds(base, ROWS_PER_SUB)

    pltpu.async_copy(src_hbm.at[wid], src_v, semi).wait()
    pltpu.async_copy(dst_hbm.at[wid], dst_v, semi).wait()

    for h in range(2):
        ch = pl.ds(h * CH, CH)
        @pl.when(cid == 0)
        def _():
            pltpu.sync_copy(cur_hbm.at[sl, ch], acc_sh.at[sl])
        @pl.when(cid != 0)
        def _():
            pltpu.sync_copy(zeros_hbm.at[sl], acc_sh.at[sl])
        pltpu.sync_copy(cur_hbm.at[sl, ch], u_sh.at[sl])
        plsc.subcore_barrier()
        _edge_pipeline(u_sh, acc_sh, src_v, dst_v, rows, gsems, ssems)
        plsc.subcore_barrier()
        pltpu.sync_copy(acc_sh.at[sl], out_hbm.at[cid, sl, ch])
        plsc.subcore_barrier()


@functools.partial(
    pl.kernel,
    out_type=[
        jax.ShapeDtypeStruct((2, NPAD, CPAD), jnp.float32),
        jax.ShapeDtypeStruct((NPAD, CPAD), jnp.float32),
    ],
    mesh=_MESH,
    scratch_types=[
        pltpu.VMEM((NB, BATCH), jnp.int32),
        pltpu.VMEM((NB, BATCH), jnp.int32),
        [pltpu.VMEM((BATCH, CH), jnp.float32)] * 8,
        [pltpu.VMEM((SLAB, CH), jnp.float32)] * 4,
        pltpu.VMEM_SHARED((NPAD, CH), jnp.float32),
        pltpu.VMEM_SHARED((NPAD, CH), jnp.float32),
        [pltpu.SemaphoreType.DMA] * 8,
        [pltpu.SemaphoreType.DMA] * 8,
        pltpu.SemaphoreType.DMA,
    ],
    compiler_params=_SC_PARAMS,
)
def _prop_fused(parts_hbm, dsq_hbm, src_hbm, dst_hbm, zeros_hbm,
                out_hbm, u_hbm,
                src_v, dst_v, rows, stg, acc_sh, u_sh, gsems, ssems, semi):
    cid = lax.axis_index("c")
    sid = lax.axis_index("s")
    wid = cid * 16 + sid
    base = sid * ROWS_PER_SUB
    sl = pl.ds(base, ROWS_PER_SUB)
    pa, pb, pd, pu = stg

    pltpu.async_copy(src_hbm.at[wid], src_v, semi).wait()
    pltpu.async_copy(dst_hbm.at[wid], dst_v, semi).wait()

    for h in range(2):
        ch = pl.ds(h * CH, CH)
        @pl.when(cid != 0)
        def _():
            pltpu.sync_copy(zeros_hbm.at[sl], acc_sh.at[sl])
        for t in range(NSLAB):
            rs = pl.ds(base + t * SLAB, SLAB)
            pltpu.sync_copy(parts_hbm.at[0, rs, ch], pa)
            pltpu.sync_copy(parts_hbm.at[1, rs, ch], pb)
            pltpu.sync_copy(dsq_hbm.at[rs, ch], pd)

            def crow(r, _):
                for c in (0, 8):
                    cs = pl.ds(c, 16)
                    pu[r, cs] = (pa[r, cs] + pb[r, cs]) * pd[r, cs]
                return 0

            lax.fori_loop(0, SLAB, crow, 0)
            pltpu.sync_copy(pu, u_sh.at[rs])
            @pl.when(cid == 0)
            def _():
                pltpu.sync_copy(pu, acc_sh.at[rs])
                pltpu.sync_copy(pu, u_hbm.at[rs, ch])
        plsc.subcore_barrier()
        _edge_pipeline(u_sh, acc_sh, src_v, dst_v, rows, gsems, ssems)
        plsc.subcore_barrier()
        pltpu.sync_copy(acc_sh.at[sl], out_hbm.at[cid, sl, ch])
        plsc.subcore_barrier()


def _mlp_body(x_ref, w1_ref, b1_ref, w2_ref, b2_ref, deg_ref,
              u0_ref, dinvsq_ref, dsq48_ref):
    h = jnp.maximum(
        jnp.dot(x_ref[...], w1_ref[...], preferred_element_type=jnp.float32)
        + b1_ref[...][None, :], 0.0)
    h = jnp.dot(h, w2_ref[...], preferred_element_type=jnp.float32) \
        + b2_ref[...][None, :]
    deg = deg_ref[0, :] + deg_ref[1, :] + 1.0
    dinv = lax.rsqrt(deg)
    u0_ref[...] = h * dinv[:, None]
    dinvsq_ref[...] = 1.0 / deg
    dsq48_ref[...] = jnp.broadcast_to((1.0 / deg)[:, None], dsq48_ref.shape)


def _mlp(x_pad, W1, b1, W2p, b2p, deg_part):
    blk = 512
    grid = NPAD // blk
    return pl.pallas_call(
        _mlp_body,
        grid=(grid,),
        in_specs=[
            pl.BlockSpec((blk, F_IN), lambda i: (i, 0)),
            pl.BlockSpec((F_IN, HID), lambda i: (0, 0)),
            pl.BlockSpec((HID,), lambda i: (0,)),
            pl.BlockSpec((HID, CPAD), lambda i: (0, 0)),
            pl.BlockSpec((CPAD,), lambda i: (0,)),
            pl.BlockSpec((2, blk), lambda i: (0, i)),
        ],
        out_specs=[
            pl.BlockSpec((blk, CPAD), lambda i: (i, 0)),
            pl.BlockSpec((blk,), lambda i: (i,)),
            pl.BlockSpec((blk, CPAD), lambda i: (i, 0)),
        ],
        out_shape=[
            jax.ShapeDtypeStruct((NPAD, CPAD), jnp.float32),
            jax.ShapeDtypeStruct((NPAD,), jnp.float32),
            jax.ShapeDtypeStruct((NPAD, CPAD), jnp.float32),
        ],
    )(x_pad, W1, b1, W2p, b2p, deg_part)


def _final_body(coefs, *refs):
    us = refs[:-3]
    parts_ref = refs[-3]
    dinvsq_ref = refs[-2]
    out_ref = refs[-1]
    acc = coefs[0] * us[0][...]
    for c, u in zip(coefs[1:-1], us[1:]):
        acc = acc + c * u[...]
    u_last = (parts_ref[0] + parts_ref[1]) * dinvsq_ref[...]
    acc = acc + coefs[-1] * u_last
    v = acc * lax.rsqrt(dinvsq_ref[...])
    col = lax.broadcasted_iota(jnp.int32, v.shape, 1)
    valid = col < CLS
    neg = jnp.full_like(v, -jnp.inf)
    m = jnp.max(jnp.where(valid, v, neg), axis=1, keepdims=True)
    ex = jnp.where(valid, jnp.exp(v - m), 0.0)
    s = jnp.sum(ex, axis=1, keepdims=True)
    res = v - m - jnp.log(s)
    out_ref[...] = res[:, :CLS]


def _final(us, parts_last, dinvsq, coefs):
    blk = 400
    grid = N // blk
    return pl.pallas_call(
        functools.partial(_final_body, coefs),
        grid=(grid,),
        in_specs=[pl.BlockSpec((blk, CPAD), lambda i: (i, 0)) for _ in us]
        + [pl.BlockSpec((2, blk, CPAD), lambda i: (0, i, 0)),
           pl.BlockSpec((blk, 1), lambda i: (i, 0))],
        out_specs=pl.BlockSpec((blk, CLS), lambda i: (i, 0)),
        out_shape=jax.ShapeDtypeStruct((N, CLS), jnp.float32),
    )(*us, parts_last, dinvsq[:, None])


def kernel(x, edge_index, W1, b1, W2, b2):
    src = edge_index[0].astype(jnp.int32)
    dst = edge_index[1].astype(jnp.int32)
    epad = NTILES * EPT - E
    src = jnp.concatenate([src, jnp.zeros((epad,), jnp.int32)])
    dst = jnp.concatenate([dst, jnp.full((epad,), NPAD - 1, jnp.int32)])
    src_t = src.reshape(NTILES, NB, BATCH)
    dst_t = dst.reshape(NTILES, NB, BATCH)

    x_pad = jnp.pad(x, ((0, NPAD - N), (0, 0)))
    W2p = jnp.pad(W2, ((0, 0), (0, CPAD - CLS)))
    b2p = jnp.pad(b2, ((0, CPAD - CLS),))

    deg_part = _deg_kernel(dst_t)
    u0, dinvsq, dsq48 = _mlp(x_pad, W1, b1, W2p, b2p, deg_part)

    zeros_pad = jnp.zeros((NPAD, CH), jnp.float32)

    khalf = K // 2
    coef = [ALPHA * (1.0 - ALPHA) ** i for i in range(khalf + 1)]
    coef[khalf] = (1.0 - ALPHA) ** khalf

    parts = _prop_first(u0, src_t, dst_t, zeros_pad)
    evens = [u0]
    for k in range(2, K + 1):
        parts, u_prev = _prop_fused(parts, dsq48, src_t, dst_t, zeros_pad)
        if k % 2 == 1:
            evens.append(u_prev)
    return _final(evens, parts, dinvsq, coef)


# speedup vs baseline: 2.5679x; 1.1018x over previous
import functools

import jax
import jax.numpy as jnp
from jax import lax
from jax.experimental import pallas as pl
from jax.experimental.pallas import tpu as pltpu
from jax.experimental.pallas import tpu_sc as plsc

N = 10000
E = 320000
F_IN = 128
HID = 64
CLS = 47
K = 10
ALPHA = 0.1

NPAD = 10240
CPAD = 48
NTILES = 32
BATCH = 128
EPT = 10240
NB = EPT // BATCH
ROWS_PER_SUB = NPAD // 16

_MESH = plsc.VectorSubcoreMesh(core_axis_name="c", subcore_axis_name="s")
_SC_PARAMS = pltpu.CompilerParams(use_tc_tiling_on_sc=False)


def _fill_f32(ref, value, total):
    vec = jnp.full((16,), value, dtype=jnp.float32)

    def body(i, _):
        ref[pl.ds(i * 16, 16)] = vec
        return 0

    lax.fori_loop(0, total // 16, body, 0)


@functools.partial(
    pl.kernel,
    out_type=jax.ShapeDtypeStruct((2, NPAD), jnp.float32),
    mesh=_MESH,
    scratch_types=[
        pltpu.VMEM((NB, BATCH), jnp.int32),
        pltpu.VMEM((BATCH,), jnp.float32),
        pltpu.VMEM((ROWS_PER_SUB,), jnp.float32),
        pltpu.VMEM_SHARED((NPAD,), jnp.float32),
        pltpu.SemaphoreType.DMA,
    ],
    compiler_params=_SC_PARAMS,
)
def _deg_kernel(dst_hbm, out_hbm, idx_v, ones_v, zeros_v, acc_sh, sem):
    cid = lax.axis_index("c")
    sid = lax.axis_index("s")
    wid = cid * 16 + sid
    _fill_f32(zeros_v, 0.0, ROWS_PER_SUB)
    _fill_f32(ones_v, 1.0, BATCH)
    pltpu.sync_copy(zeros_v, acc_sh.at[pl.ds(sid * ROWS_PER_SUB, ROWS_PER_SUB)])
    plsc.subcore_barrier()
    pltpu.async_copy(dst_hbm.at[wid], idx_v, sem).wait()

    def body(j, _):
        pltpu.sync_copy(ones_v, acc_sh.at[idx_v.at[j]], add=True)
        return 0

    lax.fori_loop(0, NB, body, 0)
    plsc.subcore_barrier()
    sl = pl.ds(sid * ROWS_PER_SUB, ROWS_PER_SUB)
    pltpu.sync_copy(acc_sh.at[sl], out_hbm.at[cid, sl])


CH = CPAD // 2
SLAB = 128
NSLAB = ROWS_PER_SUB // SLAB


def _edge_pipeline(u_sh, acc_sh, src_v, dst_v, rows, gsems, ssems):
    for j in range(4):
        pltpu.async_copy(u_sh.at[src_v.at[j]], rows[j], gsems[j])

    def body(jj, _):
        for b in range(8):
            j = jj * 8 + b
            pltpu.make_async_copy(u_sh.at[src_v.at[j]], rows[b], gsems[b]).wait()
            pltpu.async_copy(rows[b], acc_sh.at[dst_v.at[j]], ssems[b], add=True)
            @pl.when(j + 4 < NB)
            def _():
                bn = (b + 4) % 8
                @pl.when(j >= 4)
                def _():
                    pltpu.make_async_copy(
                        rows[bn], acc_sh.at[dst_v.at[j - 4]], ssems[bn]).wait()
                pltpu.async_copy(u_sh.at[src_v.at[j + 4]], rows[bn], gsems[bn])
        return 0

    lax.fori_loop(0, NB // 8, body, 0)
    for j in range(NB - 8, NB):
        b = j % 8
        pltpu.make_async_copy(rows[b], acc_sh.at[dst_v.at[j]], ssems[b]).wait()


@functools.partial(
    pl.kernel,
    out_type=jax.ShapeDtypeStruct((2, NPAD, CPAD), jnp.float32),
    mesh=_MESH,
    scratch_types=[
        pltpu.VMEM((NB, BATCH), jnp.int32),
        pltpu.VMEM((NB, BATCH), jnp.int32),
        [pltpu.VMEM((BATCH, CH), jnp.float32)] * 8,
        pltpu.VMEM_SHARED((NPAD, CH), jnp.float32),
        pltpu.VMEM_SHARED((NPAD, CH), jnp.float32),
        [pltpu.SemaphoreType.DMA] * 8,
        [pltpu.SemaphoreType.DMA] * 8,
        pltpu.SemaphoreType.DMA,
    ],
    compiler_params=_SC_PARAMS,
)
def _prop_first(cur_hbm, src_hbm, dst_hbm, zeros_hbm, out_hbm,
                src_v, dst_v, rows, acc_sh, u_sh, gsems, ssems, semi):
    cid = lax.axis_index("c")
    sid = lax.axis_index("s")
    wid = cid * 16 + sid
    base = sid * ROWS_PER_SUB
    sl = pl.ds(base, ROWS_PER_SUB)

    pltpu.async_copy(src_hbm.at[wid], src_v, semi).wait()
    pltpu.async_copy(dst_hbm.at[wid], dst_v, semi).wait()

    for h in range(2):
        ch = pl.ds(h * CH, CH)
        @pl.when(cid == 0)
        def _():
            pltpu.sync_copy(cur_hbm.at[sl, ch], acc_sh.at[sl])
        @pl.when(cid != 0)
        def _():
            pltpu.sync_copy(zeros_hbm.at[sl], acc_sh.at[sl])
        pltpu.sync_copy(cur_hbm.at[sl, ch], u_sh.at[sl])
        plsc.subcore_barrier()
        _edge_pipeline(u_sh, acc_sh, src_v, dst_v, rows, gsems, ssems)
        plsc.subcore_barrier()
        pltpu.sync_copy(acc_sh.at[sl], out_hbm.at[cid, sl, ch])
        plsc.subcore_barrier()


@functools.partial(
    pl.kernel,
    out_type=[
        jax.ShapeDtypeStruct((2, NPAD, CPAD), jnp.float32),
        jax.ShapeDtypeStruct((NPAD, CPAD), jnp.float32),
    ],
    mesh=_MESH,
    scratch_types=[
        pltpu.VMEM((NB, BATCH), jnp.int32),
        pltpu.VMEM((NB, BATCH), jnp.int32),
        [pltpu.VMEM((BATCH, CH), jnp.float32)] * 8,
        [pltpu.VMEM((SLAB, CH), jnp.float32)] * 4,
        pltpu.VMEM_SHARED((NPAD, CH), jnp.float32),
        pltpu.VMEM_SHARED((NPAD, CH), jnp.float32),
        [pltpu.SemaphoreType.DMA] * 8,
        [pltpu.SemaphoreType.DMA] * 8,
        pltpu.SemaphoreType.DMA,
    ],
    compiler_params=_SC_PARAMS,
)
def _prop_fused(parts_hbm, dsq_hbm, src_hbm, dst_hbm, zeros_hbm,
                out_hbm, u_hbm,
                src_v, dst_v, rows, stg, acc_sh, u_sh, gsems, ssems, semi):
    cid = lax.axis_index("c")
    sid = lax.axis_index("s")
    wid = cid * 16 + sid
    base = sid * ROWS_PER_SUB
    sl = pl.ds(base, ROWS_PER_SUB)
    pa, pb, pd, pu = stg

    pltpu.async_copy(src_hbm.at[wid], src_v, semi).wait()
    pltpu.async_copy(dst_hbm.at[wid], dst_v, semi).wait()

    for h in range(2):
        ch = pl.ds(h * CH, CH)
        @pl.when(cid != 0)
        def _():
            pltpu.sync_copy(zeros_hbm.at[sl], acc_sh.at[sl])
        for t in range(NSLAB):
            rs = pl.ds(base + t * SLAB, SLAB)
            pltpu.async_copy(parts_hbm.at[0, rs, ch], pa, semi)
            pltpu.async_copy(parts_hbm.at[1, rs, ch], pb, semi)
            pltpu.async_copy(dsq_hbm.at[rs, ch], pd, semi)
            pltpu.make_async_copy(parts_hbm.at[0, rs, ch], pa, semi).wait()
            pltpu.make_async_copy(parts_hbm.at[1, rs, ch], pb, semi).wait()
            pltpu.make_async_copy(dsq_hbm.at[rs, ch], pd, semi).wait()

            def crow(rr, _):
                for q in range(4):
                    r = rr * 4 + q
                    for c in (0, 8):
                        cs = pl.ds(c, 16)
                        pu[r, cs] = (pa[r, cs] + pb[r, cs]) * pd[r, cs]
                return 0

            lax.fori_loop(0, SLAB // 4, crow, 0)
            pltpu.sync_copy(pu, u_sh.at[rs])
            @pl.when(cid == 0)
            def _():
                pltpu.sync_copy(pu, acc_sh.at[rs])
                pltpu.sync_copy(pu, u_hbm.at[rs, ch])
        plsc.subcore_barrier()
        _edge_pipeline(u_sh, acc_sh, src_v, dst_v, rows, gsems, ssems)
        plsc.subcore_barrier()
        pltpu.sync_copy(acc_sh.at[sl], out_hbm.at[cid, sl, ch])
        plsc.subcore_barrier()


def _mlp_body(x_ref, w1_ref, b1_ref, w2_ref, b2_ref, deg_ref,
              u0_ref, dinvsq_ref, dsq48_ref):
    h = jnp.maximum(
        jnp.dot(x_ref[...], w1_ref[...], preferred_element_type=jnp.float32)
        + b1_ref[...][None, :], 0.0)
    h = jnp.dot(h, w2_ref[...], preferred_element_type=jnp.float32) \
        + b2_ref[...][None, :]
    deg = deg_ref[0, :] + deg_ref[1, :] + 1.0
    dinv = lax.rsqrt(deg)
    u0_ref[...] = h * dinv[:, None]
    dinvsq_ref[...] = 1.0 / deg
    dsq48_ref[...] = jnp.broadcast_to((1.0 / deg)[:, None], dsq48_ref.shape)


def _mlp(x_pad, W1, b1, W2p, b2p, deg_part):
    blk = 512
    grid = NPAD // blk
    return pl.pallas_call(
        _mlp_body,
        grid=(grid,),
        in_specs=[
            pl.BlockSpec((blk, F_IN), lambda i: (i, 0)),
            pl.BlockSpec((F_IN, HID), lambda i: (0, 0)),
            pl.BlockSpec((HID,), lambda i: (0,)),
            pl.BlockSpec((HID, CPAD), lambda i: (0, 0)),
            pl.BlockSpec((CPAD,), lambda i: (0,)),
            pl.BlockSpec((2, blk), lambda i: (0, i)),
        ],
        out_specs=[
            pl.BlockSpec((blk, CPAD), lambda i: (i, 0)),
            pl.BlockSpec((blk,), lambda i: (i,)),
            pl.BlockSpec((blk, CPAD), lambda i: (i, 0)),
        ],
        out_shape=[
            jax.ShapeDtypeStruct((NPAD, CPAD), jnp.float32),
            jax.ShapeDtypeStruct((NPAD,), jnp.float32),
            jax.ShapeDtypeStruct((NPAD, CPAD), jnp.float32),
        ],
    )(x_pad, W1, b1, W2p, b2p, deg_part)


def _final_body(coefs, *refs):
    us = refs[:-3]
    parts_ref = refs[-3]
    dinvsq_ref = refs[-2]
    out_ref = refs[-1]
    acc = coefs[0] * us[0][...]
    for c, u in zip(coefs[1:-1], us[1:]):
        acc = acc + c * u[...]
    u_last = (parts_ref[0] + parts_ref[1]) * dinvsq_ref[...]
    acc = acc + coefs[-1] * u_last
    v = acc * lax.rsqrt(dinvsq_ref[...])
    col = lax.broadcasted_iota(jnp.int32, v.shape, 1)
    valid = col < CLS
    neg = jnp.full_like(v, -jnp.inf)
    m = jnp.max(jnp.where(valid, v, neg), axis=1, keepdims=True)
    ex = jnp.where(valid, jnp.exp(v - m), 0.0)
    s = jnp.sum(ex, axis=1, keepdims=True)
    res = v - m - jnp.log(s)
    out_ref[...] = res[:, :CLS]


def _final(us, parts_last, dinvsq, coefs):
    blk = 400
    grid = N // blk
    return pl.pallas_call(
        functools.partial(_final_body, coefs),
        grid=(grid,),
        in_specs=[pl.BlockSpec((blk, CPAD), lambda i: (i, 0)) for _ in us]
        + [pl.BlockSpec((2, blk, CPAD), lambda i: (0, i, 0)),
           pl.BlockSpec((blk, 1), lambda i: (i, 0))],
        out_specs=pl.BlockSpec((blk, CLS), lambda i: (i, 0)),
        out_shape=jax.ShapeDtypeStruct((N, CLS), jnp.float32),
    )(*us, parts_last, dinvsq[:, None])


def kernel(x, edge_index, W1, b1, W2, b2):
    src = edge_index[0].astype(jnp.int32)
    dst = edge_index[1].astype(jnp.int32)
    epad = NTILES * EPT - E
    src = jnp.concatenate([src, jnp.zeros((epad,), jnp.int32)])
    dst = jnp.concatenate([dst, jnp.full((epad,), NPAD - 1, jnp.int32)])
    src_t = src.reshape(NTILES, NB, BATCH)
    dst_t = dst.reshape(NTILES, NB, BATCH)

    x_pad = jnp.pad(x, ((0, NPAD - N), (0, 0)))
    W2p = jnp.pad(W2, ((0, 0), (0, CPAD - CLS)))
    b2p = jnp.pad(b2, ((0, CPAD - CLS),))

    deg_part = _deg_kernel(dst_t)
    u0, dinvsq, dsq48 = _mlp(x_pad, W1, b1, W2p, b2p, deg_part)

    zeros_pad = jnp.zeros((NPAD, CH), jnp.float32)

    khalf = K // 2
    coef = [ALPHA * (1.0 - ALPHA) ** i for i in range(khalf + 1)]
    coef[khalf] = (1.0 - ALPHA) ** khalf

    parts = _prop_first(u0, src_t, dst_t, zeros_pad)
    evens = [u0]
    for k in range(2, K + 1):
        parts, u_prev = _prop_fused(parts, dsq48, src_t, dst_t, zeros_pad)
        if k % 2 == 1:
            evens.append(u_prev)
    return _final(evens, parts, dinvsq, coef)


# speedup vs baseline: 2.6468x; 1.0307x over previous
import functools

import jax
import jax.numpy as jnp
from jax import lax
from jax.experimental import pallas as pl
from jax.experimental.pallas import tpu as pltpu
from jax.experimental.pallas import tpu_sc as plsc

N = 10000
E = 320000
F_IN = 128
HID = 64
CLS = 47
K = 10
ALPHA = 0.1

NPAD = 10240
CPAD = 48
NTILES = 32
BATCH = 128
EPT = 10240
NB = EPT // BATCH
ROWS_PER_SUB = NPAD // 16

_MESH = plsc.VectorSubcoreMesh(core_axis_name="c", subcore_axis_name="s")
_SC_PARAMS = pltpu.CompilerParams(use_tc_tiling_on_sc=False)


def _fill_f32(ref, value, total):
    vec = jnp.full((16,), value, dtype=jnp.float32)

    def body(i, _):
        ref[pl.ds(i * 16, 16)] = vec
        return 0

    lax.fori_loop(0, total // 16, body, 0)


@functools.partial(
    pl.kernel,
    out_type=jax.ShapeDtypeStruct((2, NPAD), jnp.float32),
    mesh=_MESH,
    scratch_types=[
        pltpu.VMEM((NB, BATCH), jnp.int32),
        pltpu.VMEM((BATCH,), jnp.float32),
        pltpu.VMEM((ROWS_PER_SUB,), jnp.float32),
        pltpu.VMEM_SHARED((NPAD,), jnp.float32),
        pltpu.SemaphoreType.DMA,
    ],
    compiler_params=_SC_PARAMS,
)
def _deg_kernel(dst_hbm, out_hbm, idx_v, ones_v, zeros_v, acc_sh, sem):
    cid = lax.axis_index("c")
    sid = lax.axis_index("s")
    wid = cid * 16 + sid
    _fill_f32(zeros_v, 0.0, ROWS_PER_SUB)
    _fill_f32(ones_v, 1.0, BATCH)
    pltpu.sync_copy(zeros_v, acc_sh.at[pl.ds(sid * ROWS_PER_SUB, ROWS_PER_SUB)])
    plsc.subcore_barrier()
    pltpu.async_copy(dst_hbm.at[wid], idx_v, sem).wait()

    def body(j, _):
        pltpu.sync_copy(ones_v, acc_sh.at[idx_v.at[j]], add=True)
        return 0

    lax.fori_loop(0, NB, body, 0)
    plsc.subcore_barrier()
    sl = pl.ds(sid * ROWS_PER_SUB, ROWS_PER_SUB)
    pltpu.sync_copy(acc_sh.at[sl], out_hbm.at[cid, sl])


CH = CPAD // 2
SLAB = 128
NSLAB = ROWS_PER_SUB // SLAB


def _edge_pipeline(u_sh, acc_sh, src_v, dst_v, rows, gsems, ssems):
    for j in range(4):
        pltpu.async_copy(u_sh.at[src_v.at[j]], rows[j], gsems[j])

    def body(jj, _):
        for b in range(8):
            j = jj * 8 + b
            pltpu.make_async_copy(u_sh.at[src_v.at[j]], rows[b], gsems[b]).wait()
            pltpu.async_copy(rows[b], acc_sh.at[dst_v.at[j]], ssems[b], add=True)
            @pl.when(j + 4 < NB)
            def _():
                bn = (b + 4) % 8
                @pl.when(j >= 4)
                def _():
                    pltpu.make_async_copy(
                        rows[bn], acc_sh.at[dst_v.at[j - 4]], ssems[bn]).wait()
                pltpu.async_copy(u_sh.at[src_v.at[j + 4]], rows[bn], gsems[bn])
        return 0

    lax.fori_loop(0, NB // 8, body, 0)
    for j in range(NB - 8, NB):
        b = j % 8
        pltpu.make_async_copy(rows[b], acc_sh.at[dst_v.at[j]], ssems[b]).wait()


@functools.partial(
    pl.kernel,
    out_type=jax.ShapeDtypeStruct((2, NPAD, CPAD), jnp.float32),
    mesh=_MESH,
    scratch_types=[
        pltpu.VMEM((NB, BATCH), jnp.int32),
        pltpu.VMEM((NB, BATCH), jnp.int32),
        [pltpu.VMEM((BATCH, CH), jnp.float32)] * 8,
        pltpu.VMEM_SHARED((NPAD, CH), jnp.float32),
        pltpu.VMEM_SHARED((NPAD, CH), jnp.float32),
        [pltpu.SemaphoreType.DMA] * 8,
        [pltpu.SemaphoreType.DMA] * 8,
        pltpu.SemaphoreType.DMA,
    ],
    compiler_params=_SC_PARAMS,
)
def _prop_first(cur_hbm, src_hbm, dst_hbm, zeros_hbm, out_hbm,
                src_v, dst_v, rows, acc_sh, u_sh, gsems, ssems, semi):
    cid = lax.axis_index("c")
    sid = lax.axis_index("s")
    wid = cid * 16 + sid
    base = sid * ROWS_PER_SUB
    sl = pl.ds(base, ROWS_PER_SUB)

    pltpu.async_copy(src_hbm.at[wid], src_v, semi).wait()
    pltpu.async_copy(dst_hbm.at[wid], dst_v, semi).wait()

    for h in range(2):
        ch = pl.ds(h * CH, CH)
        @pl.when(cid == 0)
        def _():
            pltpu.sync_copy(cur_hbm.at[sl, ch], acc_sh.at[sl])
        @pl.when(cid != 0)
        def _():
            pltpu.sync_copy(zeros_hbm.at[sl], acc_sh.at[sl])
        pltpu.sync_copy(cur_hbm.at[sl, ch], u_sh.at[sl])
        plsc.subcore_barrier()
        _edge_pipeline(u_sh, acc_sh, src_v, dst_v, rows, gsems, ssems)
        plsc.subcore_barrier()
        pltpu.sync_copy(acc_sh.at[sl], out_hbm.at[cid, sl, ch])
        plsc.subcore_barrier()


@functools.partial(
    pl.kernel,
    out_type=[
        jax.ShapeDtypeStruct((2, NPAD, CPAD), jnp.float32),
        jax.ShapeDtypeStruct((NPAD, CPAD), jnp.float32),
    ],
    mesh=_MESH,
    scratch_types=[
        pltpu.VMEM((NB, BATCH), jnp.int32),
        pltpu.VMEM((NB, BATCH), jnp.int32),
        [pltpu.VMEM((BATCH, CH), jnp.float32)] * 8,
        [pltpu.VMEM((SLAB, CH), jnp.float32)] * 4,
        [pltpu.VMEM((SLAB, CH), jnp.float32)] * 3,
        pltpu.VMEM((ROWS_PER_SUB, CH), jnp.float32),
        pltpu.VMEM_SHARED((NPAD, CH), jnp.float32),
        pltpu.VMEM_SHARED((NPAD, CH), jnp.float32),
        [pltpu.SemaphoreType.DMA] * 8,
        [pltpu.SemaphoreType.DMA] * 8,
        pltpu.SemaphoreType.DMA,
        pltpu.SemaphoreType.DMA,
    ],
    compiler_params=_SC_PARAMS,
)
def _prop_fused(parts_hbm, dsq_hbm, src_hbm, dst_hbm, zeros_hbm,
                out_hbm, u_hbm,
                src_v, dst_v, rows, stg, stgb, pu2, acc_sh, u_sh,
                gsems, ssems, semi, semb):
    cid = lax.axis_index("c")
    sid = lax.axis_index("s")
    wid = cid * 16 + sid
    base = sid * ROWS_PER_SUB
    sl = pl.ds(base, ROWS_PER_SUB)
    pa, pb, pd, pu = stg
    pa2, pb2, pd2 = stgb

    pltpu.async_copy(src_hbm.at[wid], src_v, semi).wait()
    pltpu.async_copy(dst_hbm.at[wid], dst_v, semi).wait()

    ch0 = pl.ds(0, CH)
    ch1 = pl.ds(CH, CH)

    @pl.when(cid != 0)
    def _():
        pltpu.sync_copy(zeros_hbm.at[sl], acc_sh.at[sl])
    for t in range(NSLAB):
        rs = pl.ds(base + t * SLAB, SLAB)
        pltpu.async_copy(parts_hbm.at[0, rs, ch0], pa, semi)
        pltpu.async_copy(parts_hbm.at[1, rs, ch0], pb, semi)
        pltpu.async_copy(dsq_hbm.at[rs, ch0], pd, semi)
        pltpu.make_async_copy(parts_hbm.at[0, rs, ch0], pa, semi).wait()
        pltpu.make_async_copy(parts_hbm.at[1, rs, ch0], pb, semi).wait()
        pltpu.make_async_copy(dsq_hbm.at[rs, ch0], pd, semi).wait()

        def crow(rr, _):
            for q in range(4):
                r = rr * 4 + q
                for c in (0, 8):
                    cs = pl.ds(c, 16)
                    pu[r, cs] = (pa[r, cs] + pb[r, cs]) * pd[r, cs]
            return 0

        lax.fori_loop(0, SLAB // 4, crow, 0)
        pltpu.sync_copy(pu, u_sh.at[rs])
        @pl.when(cid == 0)
        def _():
            pltpu.sync_copy(pu, acc_sh.at[rs])
            pltpu.sync_copy(pu, u_hbm.at[rs, ch0])
    plsc.subcore_barrier()

    def rsb(t):
        return pl.ds(base + t * SLAB, SLAB)

    pltpu.async_copy(parts_hbm.at[0, rsb(0), ch1], pa2, semb)
    pltpu.async_copy(parts_hbm.at[1, rsb(0), ch1], pb2, semb)
    pltpu.async_copy(dsq_hbm.at[rsb(0), ch1], pd2, semb)
    for j in range(4):
        pltpu.async_copy(u_sh.at[src_v.at[j]], rows[j], gsems[j])

    def body0(jj, _):
        @pl.when(jj < NSLAB)
        def _():
            rs2 = pl.ds(base + jj * SLAB, SLAB)
            pltpu.make_async_copy(parts_hbm.at[0, rs2, ch1], pa2, semb).wait()
            pltpu.make_async_copy(parts_hbm.at[1, rs2, ch1], pb2, semb).wait()
            pltpu.make_async_copy(dsq_hbm.at[rs2, ch1], pd2, semb).wait()

            def crow2(rr, _):
                for q in range(4):
                    r = rr * 4 + q
                    for c in (0, 8):
                        cs = pl.ds(c, 16)
                        pu2[jj * SLAB + r, cs] = \
                            (pa2[r, cs] + pb2[r, cs]) * pd2[r, cs]
                return 0

            lax.fori_loop(0, SLAB // 4, crow2, 0)
            @pl.when(jj + 1 < NSLAB)
            def _():
                rs3 = pl.ds(base + (jj + 1) * SLAB, SLAB)
                pltpu.async_copy(parts_hbm.at[0, rs3, ch1], pa2, semb)
                pltpu.async_copy(parts_hbm.at[1, rs3, ch1], pb2, semb)
                pltpu.async_copy(dsq_hbm.at[rs3, ch1], pd2, semb)
        for b in range(8):
            j = jj * 8 + b
            pltpu.make_async_copy(u_sh.at[src_v.at[j]], rows[b], gsems[b]).wait()
            pltpu.async_copy(rows[b], acc_sh.at[dst_v.at[j]], ssems[b], add=True)
            @pl.when(j + 4 < NB)
            def _():
                bn = (b + 4) % 8
                @pl.when(j >= 4)
                def _():
                    pltpu.make_async_copy(
                        rows[bn], acc_sh.at[dst_v.at[j - 4]], ssems[bn]).wait()
                pltpu.async_copy(u_sh.at[src_v.at[j + 4]], rows[bn], gsems[bn])
        return 0

    lax.fori_loop(0, NB // 8, body0, 0)
    for j in range(NB - 8, NB):
        b = j % 8
        pltpu.make_async_copy(rows[b], acc_sh.at[dst_v.at[j]], ssems[b]).wait()
    plsc.subcore_barrier()
    pltpu.sync_copy(acc_sh.at[sl], out_hbm.at[cid, sl, ch0])
    plsc.subcore_barrier()

    @pl.when(cid != 0)
    def _():
        pltpu.sync_copy(zeros_hbm.at[sl], acc_sh.at[sl])
    pltpu.sync_copy(pu2, u_sh.at[sl])
    @pl.when(cid == 0)
    def _():
        pltpu.sync_copy(pu2, acc_sh.at[sl])
        pltpu.sync_copy(pu2, u_hbm.at[sl, ch1])
    plsc.subcore_barrier()
    _edge_pipeline(u_sh, acc_sh, src_v, dst_v, rows, gsems, ssems)
    plsc.subcore_barrier()
    pltpu.sync_copy(acc_sh.at[sl], out_hbm.at[cid, sl, ch1])
    plsc.subcore_barrier()


def _mlp_body(x_ref, w1_ref, b1_ref, w2_ref, b2_ref, deg_ref,
              u0_ref, dinvsq_ref, dsq48_ref):
    h = jnp.maximum(
        jnp.dot(x_ref[...], w1_ref[...], preferred_element_type=jnp.float32)
        + b1_ref[...][None, :], 0.0)
    h = jnp.dot(h, w2_ref[...], preferred_element_type=jnp.float32) \
        + b2_ref[...][None, :]
    deg = deg_ref[0, :] + deg_ref[1, :] + 1.0
    dinv = lax.rsqrt(deg)
    u0_ref[...] = h * dinv[:, None]
    dinvsq_ref[...] = 1.0 / deg
    dsq48_ref[...] = jnp.broadcast_to((1.0 / deg)[:, None], dsq48_ref.shape)


def _mlp(x_pad, W1, b1, W2p, b2p, deg_part):
    blk = 512
    grid = NPAD // blk
    return pl.pallas_call(
        _mlp_body,
        grid=(grid,),
        in_specs=[
            pl.BlockSpec((blk, F_IN), lambda i: (i, 0)),
            pl.BlockSpec((F_IN, HID), lambda i: (0, 0)),
            pl.BlockSpec((HID,), lambda i: (0,)),
            pl.BlockSpec((HID, CPAD), lambda i: (0, 0)),
            pl.BlockSpec((CPAD,), lambda i: (0,)),
            pl.BlockSpec((2, blk), lambda i: (0, i)),
        ],
        out_specs=[
            pl.BlockSpec((blk, CPAD), lambda i: (i, 0)),
            pl.BlockSpec((blk,), lambda i: (i,)),
            pl.BlockSpec((blk, CPAD), lambda i: (i, 0)),
        ],
        out_shape=[
            jax.ShapeDtypeStruct((NPAD, CPAD), jnp.float32),
            jax.ShapeDtypeStruct((NPAD,), jnp.float32),
            jax.ShapeDtypeStruct((NPAD, CPAD), jnp.float32),
        ],
    )(x_pad, W1, b1, W2p, b2p, deg_part)


def _final_body(coefs, *refs):
    us = refs[:-3]
    parts_ref = refs[-3]
    dinvsq_ref = refs[-2]
    out_ref = refs[-1]
    acc = coefs[0] * us[0][...]
    for c, u in zip(coefs[1:-1], us[1:]):
        acc = acc + c * u[...]
    u_last = (parts_ref[0] + parts_ref[1]) * dinvsq_ref[...]
    acc = acc + coefs[-1] * u_last
    v = acc * lax.rsqrt(dinvsq_ref[...])
    col = lax.broadcasted_iota(jnp.int32, v.shape, 1)
    valid = col < CLS
    neg = jnp.full_like(v, -jnp.inf)
    m = jnp.max(jnp.where(valid, v, neg), axis=1, keepdims=True)
    ex = jnp.where(valid, jnp.exp(v - m), 0.0)
    s = jnp.sum(ex, axis=1, keepdims=True)
    res = v - m - jnp.log(s)
    out_ref[...] = res[:, :CLS]


def _final(us, parts_last, dinvsq, coefs):
    blk = 400
    grid = N // blk
    return pl.pallas_call(
        functools.partial(_final_body, coefs),
        grid=(grid,),
        in_specs=[pl.BlockSpec((blk, CPAD), lambda i: (i, 0)) for _ in us]
        + [pl.BlockSpec((2, blk, CPAD), lambda i: (0, i, 0)),
           pl.BlockSpec((blk, 1), lambda i: (i, 0))],
        out_specs=pl.BlockSpec((blk, CLS), lambda i: (i, 0)),
        out_shape=jax.ShapeDtypeStruct((N, CLS), jnp.float32),
    )(*us, parts_last, dinvsq[:, None])


def kernel(x, edge_index, W1, b1, W2, b2):
    src = edge_index[0].astype(jnp.int32)
    dst = edge_index[1].astype(jnp.int32)
    epad = NTILES * EPT - E
    src = jnp.concatenate([src, jnp.zeros((epad,), jnp.int32)])
    dst = jnp.concatenate([dst, jnp.full((epad,), NPAD - 1, jnp.int32)])
    src_t = src.reshape(NTILES, NB, BATCH)
    dst_t = dst.reshape(NTILES, NB, BATCH)

    x_pad = jnp.pad(x, ((0, NPAD - N), (0, 0)))
    W2p = jnp.pad(W2, ((0, 0), (0, CPAD - CLS)))
    b2p = jnp.pad(b2, ((0, CPAD - CLS),))

    deg_part = _deg_kernel(dst_t)
    u0, dinvsq, dsq48 = _mlp(x_pad, W1, b1, W2p, b2p, deg_part)

    zeros_pad = jnp.zeros((NPAD, CH), jnp.float32)

    khalf = K // 2
    coef = [ALPHA * (1.0 - ALPHA) ** i for i in range(khalf + 1)]
    coef[khalf] = (1.0 - ALPHA) ** khalf

    parts = _prop_first(u0, src_t, dst_t, zeros_pad)
    evens = [u0]
    for k in range(2, K + 1):
        parts, u_prev = _prop_fused(parts, dsq48, src_t, dst_t, zeros_pad)
        if k % 2 == 1:
            evens.append(u_prev)
    return _final(evens, parts, dinvsq, coef)
